# baseline (speedup 1.0000x reference)
import functools
import math

import numpy as np

import jax
import jax.numpy as jnp
from jax import lax
from jax.experimental import pallas as pl
from jax.experimental.pallas import tpu as pltpu
from jax.experimental.pallas import tpu_sc as plsc

B = 2
LQ = 2048
D = 256
NH = 8
HD = 32
P = 25
HF = 180
WF = 180
LV = HF * WF
R = 4 * P
NW = 32
QW = (B * NH * LQ) // NW
CQ = 8
NCHUNK = QW // CQ

_ILV = np.empty((HD,), dtype=np.int64)
_ILV[0::2] = np.arange(HD // 2)
_ILV[1::2] = np.arange(HD // 2, HD)
_VPERM = np.concatenate([h * HD + _ILV for h in range(NH)])


def _prep_body(q_ref, rw_ref, wa_ref, ba_ref, wb_ref, bb_ref, kx_ref, ky_ref,
               vr_ref, attn_ref, idx_ref, w_ref):
    i = pl.program_id(0)
    off = (i // NH) * (LV * NH) + (i % NH)

    qb = q_ref[0]
    wa = wa_ref[0]
    logits = lax.dot_general(qb, wa, (((1,), (1,)), ((), ())),
                             preferred_element_type=jnp.float32)
    logits = logits + ba_ref[0]
    m = jnp.max(logits, axis=-1, keepdims=True)
    e = jnp.exp(logits - m)
    attn = e / jnp.sum(e, axis=-1, keepdims=True)
    attn_ref[0, :, 0, 0] = attn

    ob = lax.dot_general(qb, wb_ref[0], (((1,), (1,)), ((), ())),
                         preferred_element_type=jnp.float32)
    ob = ob + bb_ref[0]
    rw = rw_ref[0]
    cx, cy = rw[:, 0:1], rw[:, 1:2]
    bw, bh = rw[:, 3:4], rw[:, 4:5]
    ang = rw[:, 6:7]
    dx, dy = ob[:, 0:1], ob[:, 1:2]
    dw, dh = ob[:, 2:3], ob[:, 3:4]
    da = ob[:, 4:5]

    angle = (ang + da * (1.0 / 16.0)) * (2.0 * math.pi)
    cosa = jnp.cos(angle)
    sina = jnp.sin(angle)
    ctr_x = cx + dx * (1.0 / 8.0) * bw
    ctr_y = cy + dy * (1.0 / 8.0) * bh
    sw = jnp.maximum(bw + dw * (1.0 / 8.0) * bw, 0.0)
    sh = jnp.maximum(bh + dh * (1.0 / 8.0) * bh, 0.0)

    gx = kx_ref[...] * sw
    gy = ky_ref[...] * sh
    vrx = vr_ref[0, 0:1, 0:1]
    vry = vr_ref[0, 0:1, 1:2]
    grid_x = (ctr_x + gx * cosa - gy * sina) * vrx
    grid_y = (ctr_y + gx * sina + gy * cosa) * vry

    x = grid_x * WF - 0.5
    y = grid_y * HF - 0.5
    x0 = jnp.floor(x)
    y0 = jnp.floor(y)
    lx = x - x0
    ly = y - y0
    x0i = x0.astype(jnp.int32)
    y0i = y0.astype(jnp.int32)

    def corner(yi, xi, wbil):
        valid = ((yi >= 0) & (yi < HF) & (xi >= 0) & (xi < WF))
        lin = jnp.clip(yi, 0, HF - 1) * WF + jnp.clip(xi, 0, WF - 1)
        gidx = lin * NH + off
        wc = attn * wbil * valid.astype(jnp.float32)
        return gidx, wc

    i00, w00 = corner(y0i, x0i, (1.0 - ly) * (1.0 - lx))
    i01, w01 = corner(y0i, x0i + 1, (1.0 - ly) * lx)
    i10, w10 = corner(y0i + 1, x0i, ly * (1.0 - lx))
    i11, w11 = corner(y0i + 1, x0i + 1, ly * lx)

    idx_ref[0] = jnp.concatenate([i00, i01, i10, i11], axis=1)
    w_ref[0] = jnp.concatenate([w00, w01, w10, w11], axis=1)


def _prep_call(query, ref_windows, waT, ba, wbx, bbx, kx, ky, vr):
    grid = (B * NH,)
    return pl.pallas_call(
        _prep_body,
        grid=grid,
        in_specs=[
            pl.BlockSpec((1, LQ, D), lambda i: (i // NH, 0, 0)),
            pl.BlockSpec((1, LQ, 7), lambda i: (i // NH, 0, 0)),
            pl.BlockSpec((1, P, D), lambda i: (i % NH, 0, 0)),
            pl.BlockSpec((1, 1, P), lambda i: (i % NH, 0, 0)),
            pl.BlockSpec((1, 5, D), lambda i: (i % NH, 0, 0)),
            pl.BlockSpec((1, 1, 5), lambda i: (i % NH, 0, 0)),
            pl.BlockSpec((1, P), lambda i: (0, 0)),
            pl.BlockSpec((1, P), lambda i: (0, 0)),
            pl.BlockSpec((1, 1, 2), lambda i: (i // NH, 0, 0)),
        ],
        out_specs=[
            pl.BlockSpec((1, LQ, 1, 1, P), lambda i: (i // NH, 0, i % NH, 0, 0)),
            pl.BlockSpec((1, LQ, R), lambda i: (i, 0, 0)),
            pl.BlockSpec((1, LQ, R), lambda i: (i, 0, 0)),
        ],
        out_shape=[
            jax.ShapeDtypeStruct((B, LQ, NH, 1, P), jnp.float32),
            jax.ShapeDtypeStruct((B * NH, LQ, R), jnp.int32),
            jax.ShapeDtypeStruct((B * NH, LQ, R), jnp.float32),
        ],
    )(query, ref_windows, waT, ba, wbx, bbx, kx, ky, vr)


_VCH = 3600


def _vproj_body(v_ref, wv_ref, bv_ref, m_ref, o_ref):
    acc = lax.dot_general(v_ref[0], wv_ref[...], (((1,), (1,)), ((), ())),
                          preferred_element_type=jnp.float32)
    o_ref[0] = ((acc + bv_ref[...]) * (1.0 - m_ref[0])).astype(jnp.bfloat16)


def _vproj_call(value, Wv, bv2, maskf):
    grid = (B, LV // _VCH)
    return pl.pallas_call(
        _vproj_body,
        grid=grid,
        in_specs=[
            pl.BlockSpec((1, _VCH, D), lambda b, r: (b, r, 0)),
            pl.BlockSpec((D, D), lambda b, r: (0, 0)),
            pl.BlockSpec((1, D), lambda b, r: (0, 0)),
            pl.BlockSpec((1, _VCH, 1), lambda b, r: (b, r, 0)),
        ],
        out_specs=pl.BlockSpec((1, _VCH, D), lambda b, r: (b, r, 0)),
        out_shape=jax.ShapeDtypeStruct((B, LV, D), jnp.bfloat16),
    )(value, Wv, bv2, maskf)


def _sc_gather_combine(table, idxg, wg):
    mesh = plsc.VectorSubcoreMesh(core_axis_name="c", subcore_axis_name="s")

    @functools.partial(
        pl.kernel,
        mesh=mesh,
        out_type=jax.ShapeDtypeStruct((B, LQ, NH, HD), jnp.float32),
        scratch_types=[
            pltpu.VMEM((2, CQ, R), jnp.int32),
            pltpu.VMEM((2, CQ, R), jnp.float32),
            pltpu.VMEM((2, CQ, R, HD // 2), jnp.int32),
            pltpu.VMEM((CQ, HD), jnp.float32),
            pltpu.SemaphoreType.DMA,
            pltpu.SemaphoreType.DMA,
        ],
        compiler_params=pltpu.CompilerParams(use_tc_tiling_on_sc=False),
    )
    def sc_k(table_h, idx_h, w_h, out_h, idx_v, w_v, rows_v, out_v, sem0, sem1):
        sems = (sem0, sem1)
        wid = lax.axis_index("c") * 16 + lax.axis_index("s")
        b = wid // (NW // B)
        h = (wid % (NW // B)) // 2
        q0 = (wid % 2) * QW

        def stage(gg, s):
            n0 = wid * QW + gg * CQ
            pltpu.sync_copy(idx_h.at[pl.ds(n0, CQ)], idx_v.at[s])
            pltpu.sync_copy(w_h.at[pl.ds(n0, CQ)], w_v.at[s])
            for qi in range(CQ):
                pltpu.async_copy(table_h.at[idx_v.at[s, qi]],
                                 rows_v.at[s, qi], sems[s])

        def consume(gg, s):
            for qi in range(CQ):
                pltpu.make_async_copy(table_h.at[idx_v.at[s, qi]],
                                      rows_v.at[s, qi], sems[s]).wait()

            def q_body(qi, carry2):
                acc0 = jnp.zeros((16,), jnp.float32)
                acc1 = jnp.zeros((16,), jnp.float32)
                for grp in range(7):
                    base = 16 * grp if grp < 6 else R - 16
                    jstart = 0 if grp < 6 else 16 * 7 - R
                    wvec = w_v[s, qi, pl.ds(base, 16)]
                    for j in range(jstart, 16):
                        r = base + j
                        wb = jnp.full((16,), wvec[j], jnp.float32)
                        row = rows_v[s, qi, r, pl.ds(0, HD // 2)]
                        lo = lax.bitcast_convert_type(
                            row << 16, jnp.float32)
                        hi = lax.bitcast_convert_type(
                            row & jnp.int32(-65536), jnp.float32)
                        acc0 = acc0 + wb * lo
                        acc1 = acc1 + wb * hi
                out_v[qi, pl.ds(0, 16)] = acc0
                out_v[qi, pl.ds(16, 16)] = acc1
                return carry2

            lax.fori_loop(0, CQ, q_body, 0)
            pltpu.sync_copy(out_v, out_h.at[b, pl.ds(q0 + gg * CQ, CQ), h])

        stage(0, 0)

        def outer(i, carry):
            g = i * 2
            for s2 in range(2):
                gg = g + s2

                @pl.when(gg + 1 < NCHUNK)
                def _():
                    stage(gg + 1, (s2 + 1) % 2)

                consume(gg, s2)
            return carry

        lax.fori_loop(0, NCHUNK // 2, outer, 0)

    return sc_k(table, idxg, wg)


def _oproj_body(x_ref, wo_ref, bo_ref, o_ref):
    acc = lax.dot_general(x_ref[0], wo_ref[...], (((1,), (1,)), ((), ())),
                          preferred_element_type=jnp.float32)
    o_ref[0] = acc + bo_ref[...]


def _oproj_call(x, Wo, bo2):
    return pl.pallas_call(
        _oproj_body,
        grid=(B,),
        in_specs=[
            pl.BlockSpec((1, LQ, D), lambda b: (b, 0, 0)),
            pl.BlockSpec((D, D), lambda b: (0, 0)),
            pl.BlockSpec((1, D), lambda b: (0, 0)),
        ],
        out_specs=pl.BlockSpec((1, LQ, D), lambda b: (b, 0, 0)),
        out_shape=jax.ShapeDtypeStruct((B, LQ, D), jnp.float32),
    )(x, Wo, bo2)


def kernel(query, value, v_shape, v_mask, v_start_index, v_valid_ratios,
           ref_windows, Wv, bv, Wo, bo, Wbox, bbox, Wattn, battn,
           kernel_indices):
    waT = Wattn.reshape(NH, P, D)
    ba = battn.reshape(NH, 1, P)
    wbx = Wbox.reshape(NH, 5, D)
    bbx = bbox.reshape(NH, 1, 5)
    kx = kernel_indices[:, 0].reshape(1, P)
    ky = kernel_indices[:, 1].reshape(1, P)
    vr = v_valid_ratios.reshape(B, 1, 2)
    maskf = v_mask.astype(jnp.float32).reshape(B, LV, 1)

    attn4, idxg, wg = _prep_call(query, ref_windows, waT, ba, wbx, bbx,
                                 kx, ky, vr)
    vproj = _vproj_call(value, Wv[_VPERM], bv[_VPERM].reshape(1, D), maskf)
    table = lax.bitcast_convert_type(
        vproj.reshape(B * LV * NH * (HD // 2), 2), jnp.int32
    ).reshape(B * LV * NH, HD // 2)
    sc_out = _sc_gather_combine(table, idxg.reshape(B * NH * LQ, R),
                                wg.reshape(B * NH * LQ, R))
    out = _oproj_call(sc_out.reshape(B, LQ, D), Wo, bo.reshape(1, D))
    attn_weights = attn4.reshape(B, LQ, NH, 1, 5, 5)
    return out, attn_weights


# speedup vs baseline: 10.0436x; 10.0436x over previous
import functools
import math

import numpy as np

import jax
import jax.numpy as jnp
from jax import lax
from jax.experimental import pallas as pl
from jax.experimental.pallas import tpu as pltpu
from jax.experimental.pallas import tpu_sc as plsc

B = 2
LQ = 2048
D = 256
NH = 8
HD = 32
P = 25
HF = 180
WF = 180
LV = HF * WF
R = 4 * P
NW = 32
QW = (B * NH * LQ) // NW
CQ = 8
NCHUNK = QW // CQ

_VPERM = np.concatenate([
    np.concatenate([h * HD + np.arange(HD // 2) for h in range(NH)]),
    np.concatenate([h * HD + HD // 2 + np.arange(HD // 2) for h in range(NH)]),
])


def _prep_body(q_ref, rw_ref, wa_ref, ba_ref, wb_ref, bb_ref, kx_ref, ky_ref,
               vr_ref, attn_ref, idx_ref, w_ref):
    i = pl.program_id(0)
    off = (i // NH) * (LV * NH) + (i % NH)

    qb = q_ref[0]
    wa = wa_ref[0]
    logits = lax.dot_general(qb, wa, (((1,), (1,)), ((), ())),
                             preferred_element_type=jnp.float32)
    logits = logits + ba_ref[0]
    m = jnp.max(logits, axis=-1, keepdims=True)
    e = jnp.exp(logits - m)
    attn = e / jnp.sum(e, axis=-1, keepdims=True)
    attn_ref[0, :, 0, 0] = attn

    ob = lax.dot_general(qb, wb_ref[0], (((1,), (1,)), ((), ())),
                         preferred_element_type=jnp.float32)
    ob = ob + bb_ref[0]
    rw = rw_ref[0]
    cx, cy = rw[:, 0:1], rw[:, 1:2]
    bw, bh = rw[:, 3:4], rw[:, 4:5]
    ang = rw[:, 6:7]
    dx, dy = ob[:, 0:1], ob[:, 1:2]
    dw, dh = ob[:, 2:3], ob[:, 3:4]
    da = ob[:, 4:5]

    angle = (ang + da * (1.0 / 16.0)) * (2.0 * math.pi)
    cosa = jnp.cos(angle)
    sina = jnp.sin(angle)
    ctr_x = cx + dx * (1.0 / 8.0) * bw
    ctr_y = cy + dy * (1.0 / 8.0) * bh
    sw = jnp.maximum(bw + dw * (1.0 / 8.0) * bw, 0.0)
    sh = jnp.maximum(bh + dh * (1.0 / 8.0) * bh, 0.0)

    gx = kx_ref[...] * sw
    gy = ky_ref[...] * sh
    vrx = vr_ref[0, 0:1, 0:1]
    vry = vr_ref[0, 0:1, 1:2]
    grid_x = (ctr_x + gx * cosa - gy * sina) * vrx
    grid_y = (ctr_y + gx * sina + gy * cosa) * vry

    x = grid_x * WF - 0.5
    y = grid_y * HF - 0.5
    x0 = jnp.floor(x)
    y0 = jnp.floor(y)
    lx = x - x0
    ly = y - y0
    x0i = x0.astype(jnp.int32)
    y0i = y0.astype(jnp.int32)

    def corner(yi, xi, wbil):
        valid = ((yi >= 0) & (yi < HF) & (xi >= 0) & (xi < WF))
        lin = jnp.clip(yi, 0, HF - 1) * WF + jnp.clip(xi, 0, WF - 1)
        gidx = lin * NH + off
        wc = attn * wbil * valid.astype(jnp.float32)
        return gidx, wc

    i00, w00 = corner(y0i, x0i, (1.0 - ly) * (1.0 - lx))
    i01, w01 = corner(y0i, x0i + 1, (1.0 - ly) * lx)
    i10, w10 = corner(y0i + 1, x0i, ly * (1.0 - lx))
    i11, w11 = corner(y0i + 1, x0i + 1, ly * lx)

    idx_ref[0] = jnp.concatenate([i00, i01, i10, i11], axis=1)
    w_ref[0] = jnp.concatenate([w00, w01, w10, w11], axis=1)


def _prep_call(query, ref_windows, waT, ba, wbx, bbx, kx, ky, vr):
    grid = (B * NH,)
    return pl.pallas_call(
        _prep_body,
        grid=grid,
        in_specs=[
            pl.BlockSpec((1, LQ, D), lambda i: (i // NH, 0, 0)),
            pl.BlockSpec((1, LQ, 7), lambda i: (i // NH, 0, 0)),
            pl.BlockSpec((1, P, D), lambda i: (i % NH, 0, 0)),
            pl.BlockSpec((1, 1, P), lambda i: (i % NH, 0, 0)),
            pl.BlockSpec((1, 5, D), lambda i: (i % NH, 0, 0)),
            pl.BlockSpec((1, 1, 5), lambda i: (i % NH, 0, 0)),
            pl.BlockSpec((1, P), lambda i: (0, 0)),
            pl.BlockSpec((1, P), lambda i: (0, 0)),
            pl.BlockSpec((1, 1, 2), lambda i: (i // NH, 0, 0)),
        ],
        out_specs=[
            pl.BlockSpec((1, LQ, 1, 1, P), lambda i: (i // NH, 0, i % NH, 0, 0)),
            pl.BlockSpec((1, LQ, R), lambda i: (i, 0, 0)),
            pl.BlockSpec((1, LQ, R), lambda i: (i, 0, 0)),
        ],
        out_shape=[
            jax.ShapeDtypeStruct((B, LQ, NH, 1, P), jnp.float32),
            jax.ShapeDtypeStruct((B * NH, LQ, R), jnp.int32),
            jax.ShapeDtypeStruct((B * NH, LQ, R), jnp.float32),
        ],
    )(query, ref_windows, waT, ba, wbx, bbx, kx, ky, vr)


_VCH = 3600


def _vproj_body(v_ref, wv_ref, bv_ref, m_ref, o_ref):
    acc = lax.dot_general(v_ref[0], wv_ref[...], (((1,), (1,)), ((), ())),
                          preferred_element_type=jnp.float32)
    acc = (acc + bv_ref[...]) * (1.0 - m_ref[0])
    ri = lax.bitcast_convert_type(acc, jnp.int32)
    rnd = ri + jnp.int32(0x7FFF) + ((ri >> 16) & jnp.int32(1))
    lo = (rnd[:, : D // 2] >> 16) & jnp.int32(0xFFFF)
    hi = rnd[:, D // 2:] & jnp.int32(-65536)
    o_ref[0] = lo | hi


def _vproj_call(value, Wv, bv2, maskf):
    grid = (B, LV // _VCH)
    return pl.pallas_call(
        _vproj_body,
        grid=grid,
        in_specs=[
            pl.BlockSpec((1, _VCH, D), lambda b, r: (b, r, 0)),
            pl.BlockSpec((D, D), lambda b, r: (0, 0)),
            pl.BlockSpec((1, D), lambda b, r: (0, 0)),
            pl.BlockSpec((1, _VCH, 1), lambda b, r: (b, r, 0)),
        ],
        out_specs=pl.BlockSpec((1, _VCH, D // 2), lambda b, r: (b, r, 0)),
        out_shape=jax.ShapeDtypeStruct((B, LV, D // 2), jnp.int32),
    )(value, Wv, bv2, maskf)


def _sc_gather_combine(table, idxg, wg):
    mesh = plsc.VectorSubcoreMesh(core_axis_name="c", subcore_axis_name="s")

    @functools.partial(
        pl.kernel,
        mesh=mesh,
        out_type=jax.ShapeDtypeStruct((B, LQ, NH, HD), jnp.float32),
        scratch_types=[
            pltpu.VMEM((2, CQ, R), jnp.int32),
            pltpu.VMEM((2, CQ, R), jnp.float32),
            pltpu.VMEM((2, CQ, R, HD // 2), jnp.int32),
            pltpu.VMEM((CQ, HD), jnp.float32),
            pltpu.SemaphoreType.DMA,
            pltpu.SemaphoreType.DMA,
        ],
        compiler_params=pltpu.CompilerParams(use_tc_tiling_on_sc=False),
    )
    def sc_k(table_h, idx_h, w_h, out_h, idx_v, w_v, rows_v, out_v, sem0, sem1):
        sems = (sem0, sem1)
        wid = lax.axis_index("c") * 16 + lax.axis_index("s")
        b = wid // (NW // B)
        h = (wid % (NW // B)) // 2
        q0 = (wid % 2) * QW

        def stage(gg, s):
            n0 = wid * QW + gg * CQ
            pltpu.sync_copy(idx_h.at[pl.ds(n0, CQ)], idx_v.at[s])
            pltpu.sync_copy(w_h.at[pl.ds(n0, CQ)], w_v.at[s])
            for qi in range(CQ):
                pltpu.async_copy(table_h.at[idx_v.at[s, qi]],
                                 rows_v.at[s, qi], sems[s])

        def consume(gg, s):
            for qi in range(CQ):
                pltpu.make_async_copy(table_h.at[idx_v.at[s, qi]],
                                      rows_v.at[s, qi], sems[s]).wait()

            def q_body(qi, carry2):
                acc0 = jnp.zeros((16,), jnp.float32)
                acc1 = jnp.zeros((16,), jnp.float32)
                for grp in range(7):
                    base = 16 * grp if grp < 6 else R - 16
                    jstart = 0 if grp < 6 else 16 * 7 - R
                    wvec = w_v[s, qi, pl.ds(base, 16)]
                    for j in range(jstart, 16):
                        r = base + j
                        wb = jnp.full((16,), wvec[j], jnp.float32)
                        row = rows_v[s, qi, r, pl.ds(0, HD // 2)]
                        lo = lax.bitcast_convert_type(
                            row << 16, jnp.float32)
                        hi = lax.bitcast_convert_type(
                            row & jnp.int32(-65536), jnp.float32)
                        acc0 = acc0 + wb * lo
                        acc1 = acc1 + wb * hi
                out_v[qi, pl.ds(0, 16)] = acc0
                out_v[qi, pl.ds(16, 16)] = acc1
                return carry2

            lax.fori_loop(0, CQ, q_body, 0)
            pltpu.sync_copy(out_v, out_h.at[b, pl.ds(q0 + gg * CQ, CQ), h])

        stage(0, 0)

        def outer(i, carry):
            g = i * 2
            for s2 in range(2):
                gg = g + s2

                @pl.when(gg + 1 < NCHUNK)
                def _():
                    stage(gg + 1, (s2 + 1) % 2)

                consume(gg, s2)
            return carry

        lax.fori_loop(0, NCHUNK // 2, outer, 0)

    return sc_k(table, idxg, wg)


def _oproj_body(x_ref, wo_ref, bo_ref, o_ref):
    acc = lax.dot_general(x_ref[0], wo_ref[...], (((1,), (1,)), ((), ())),
                          preferred_element_type=jnp.float32)
    o_ref[0] = acc + bo_ref[...]


def _oproj_call(x, Wo, bo2):
    return pl.pallas_call(
        _oproj_body,
        grid=(B,),
        in_specs=[
            pl.BlockSpec((1, LQ, D), lambda b: (b, 0, 0)),
            pl.BlockSpec((D, D), lambda b: (0, 0)),
            pl.BlockSpec((1, D), lambda b: (0, 0)),
        ],
        out_specs=pl.BlockSpec((1, LQ, D), lambda b: (b, 0, 0)),
        out_shape=jax.ShapeDtypeStruct((B, LQ, D), jnp.float32),
    )(x, Wo, bo2)


def kernel(query, value, v_shape, v_mask, v_start_index, v_valid_ratios,
           ref_windows, Wv, bv, Wo, bo, Wbox, bbox, Wattn, battn,
           kernel_indices):
    waT = Wattn.reshape(NH, P, D)
    ba = battn.reshape(NH, 1, P)
    wbx = Wbox.reshape(NH, 5, D)
    bbx = bbox.reshape(NH, 1, 5)
    kx = kernel_indices[:, 0].reshape(1, P)
    ky = kernel_indices[:, 1].reshape(1, P)
    vr = v_valid_ratios.reshape(B, 1, 2)
    maskf = v_mask.astype(jnp.float32).reshape(B, LV, 1)

    attn4, idxg, wg = _prep_call(query, ref_windows, waT, ba, wbx, bbx,
                                 kx, ky, vr)
    vproj = _vproj_call(value, Wv[_VPERM], bv[_VPERM].reshape(1, D), maskf)
    table = vproj.reshape(B * LV * NH, HD // 2)
    sc_out = _sc_gather_combine(table, idxg.reshape(B * NH * LQ, R),
                                wg.reshape(B * NH * LQ, R))
    out = _oproj_call(sc_out.reshape(B, LQ, D), Wo, bo.reshape(1, D))
    attn_weights = attn4.reshape(B, LQ, NH, 1, 5, 5)
    return out, attn_weights


# speedup vs baseline: 10.9624x; 1.0915x over previous
import functools
import math

import numpy as np

import jax
import jax.numpy as jnp
from jax import lax
from jax.experimental import pallas as pl
from jax.experimental.pallas import tpu as pltpu
from jax.experimental.pallas import tpu_sc as plsc

B = 2
LQ = 2048
D = 256
NH = 8
HD = 32
P = 25
HF = 180
WF = 180
LV = HF * WF
R = 4 * P
NW = 32
QW = (B * NH * LQ) // NW
CQ = 16
NCHUNK = QW // CQ

_VPERM = np.concatenate([
    np.concatenate([h * HD + np.arange(HD // 2) for h in range(NH)]),
    np.concatenate([h * HD + HD // 2 + np.arange(HD // 2) for h in range(NH)]),
])


def _prep_body(q_ref, rw_ref, wa_ref, ba_ref, wb_ref, bb_ref, kx_ref, ky_ref,
               vr_ref, attn_ref, idx_ref, w_ref):
    i = pl.program_id(0)
    off = (i // NH) * (LV * NH) + (i % NH)

    qb = q_ref[0]
    wa = wa_ref[0]
    logits = lax.dot_general(qb, wa, (((1,), (1,)), ((), ())),
                             preferred_element_type=jnp.float32)
    logits = logits + ba_ref[0]
    m = jnp.max(logits, axis=-1, keepdims=True)
    e = jnp.exp(logits - m)
    attn = e / jnp.sum(e, axis=-1, keepdims=True)
    attn_ref[0, :, 0, 0] = attn

    ob = lax.dot_general(qb, wb_ref[0], (((1,), (1,)), ((), ())),
                         preferred_element_type=jnp.float32)
    ob = ob + bb_ref[0]
    rw = rw_ref[0]
    cx, cy = rw[:, 0:1], rw[:, 1:2]
    bw, bh = rw[:, 3:4], rw[:, 4:5]
    ang = rw[:, 6:7]
    dx, dy = ob[:, 0:1], ob[:, 1:2]
    dw, dh = ob[:, 2:3], ob[:, 3:4]
    da = ob[:, 4:5]

    angle = (ang + da * (1.0 / 16.0)) * (2.0 * math.pi)
    cosa = jnp.cos(angle)
    sina = jnp.sin(angle)
    ctr_x = cx + dx * (1.0 / 8.0) * bw
    ctr_y = cy + dy * (1.0 / 8.0) * bh
    sw = jnp.maximum(bw + dw * (1.0 / 8.0) * bw, 0.0)
    sh = jnp.maximum(bh + dh * (1.0 / 8.0) * bh, 0.0)

    gx = kx_ref[...] * sw
    gy = ky_ref[...] * sh
    vrx = vr_ref[0, 0:1, 0:1]
    vry = vr_ref[0, 0:1, 1:2]
    grid_x = (ctr_x + gx * cosa - gy * sina) * vrx
    grid_y = (ctr_y + gx * sina + gy * cosa) * vry

    x = grid_x * WF - 0.5
    y = grid_y * HF - 0.5
    x0 = jnp.floor(x)
    y0 = jnp.floor(y)
    lx = x - x0
    ly = y - y0
    x0i = x0.astype(jnp.int32)
    y0i = y0.astype(jnp.int32)

    def corner(yi, xi, wbil):
        valid = ((yi >= 0) & (yi < HF) & (xi >= 0) & (xi < WF))
        lin = jnp.clip(yi, 0, HF - 1) * WF + jnp.clip(xi, 0, WF - 1)
        gidx = lin * NH + off
        wc = attn * wbil * valid.astype(jnp.float32)
        return gidx, wc

    i00, w00 = corner(y0i, x0i, (1.0 - ly) * (1.0 - lx))
    i01, w01 = corner(y0i, x0i + 1, (1.0 - ly) * lx)
    i10, w10 = corner(y0i + 1, x0i, ly * (1.0 - lx))
    i11, w11 = corner(y0i + 1, x0i + 1, ly * lx)

    idx_ref[0] = jnp.concatenate([i00, i01, i10, i11], axis=1)
    w_ref[0] = jnp.concatenate([w00, w01, w10, w11], axis=1)


def _prep_call(query, ref_windows, waT, ba, wbx, bbx, kx, ky, vr):
    grid = (B * NH,)
    return pl.pallas_call(
        _prep_body,
        grid=grid,
        in_specs=[
            pl.BlockSpec((1, LQ, D), lambda i: (i // NH, 0, 0)),
            pl.BlockSpec((1, LQ, 7), lambda i: (i // NH, 0, 0)),
            pl.BlockSpec((1, P, D), lambda i: (i % NH, 0, 0)),
            pl.BlockSpec((1, 1, P), lambda i: (i % NH, 0, 0)),
            pl.BlockSpec((1, 5, D), lambda i: (i % NH, 0, 0)),
            pl.BlockSpec((1, 1, 5), lambda i: (i % NH, 0, 0)),
            pl.BlockSpec((1, P), lambda i: (0, 0)),
            pl.BlockSpec((1, P), lambda i: (0, 0)),
            pl.BlockSpec((1, 1, 2), lambda i: (i // NH, 0, 0)),
        ],
        out_specs=[
            pl.BlockSpec((1, LQ, 1, 1, P), lambda i: (i // NH, 0, i % NH, 0, 0)),
            pl.BlockSpec((1, LQ, R), lambda i: (i, 0, 0)),
            pl.BlockSpec((1, LQ, R), lambda i: (i, 0, 0)),
        ],
        out_shape=[
            jax.ShapeDtypeStruct((B, LQ, NH, 1, P), jnp.float32),
            jax.ShapeDtypeStruct((B * NH, LQ, R), jnp.int32),
            jax.ShapeDtypeStruct((B * NH, LQ, R), jnp.float32),
        ],
    )(query, ref_windows, waT, ba, wbx, bbx, kx, ky, vr)


_VCH = 3600


def _vproj_body(v_ref, wv_ref, bv_ref, m_ref, o_ref):
    acc = lax.dot_general(v_ref[0], wv_ref[...], (((1,), (1,)), ((), ())),
                          preferred_element_type=jnp.float32)
    acc = (acc + bv_ref[...]) * (1.0 - m_ref[0])
    ri = lax.bitcast_convert_type(acc, jnp.int32)
    rnd = ri + jnp.int32(0x7FFF) + ((ri >> 16) & jnp.int32(1))
    lo = (rnd[:, : D // 2] >> 16) & jnp.int32(0xFFFF)
    hi = rnd[:, D // 2:] & jnp.int32(-65536)
    o_ref[0] = lo | hi


def _vproj_call(value, Wv, bv2, maskf):
    grid = (B, LV // _VCH)
    return pl.pallas_call(
        _vproj_body,
        grid=grid,
        in_specs=[
            pl.BlockSpec((1, _VCH, D), lambda b, r: (b, r, 0)),
            pl.BlockSpec((D, D), lambda b, r: (0, 0)),
            pl.BlockSpec((1, D), lambda b, r: (0, 0)),
            pl.BlockSpec((1, _VCH, 1), lambda b, r: (b, r, 0)),
        ],
        out_specs=pl.BlockSpec((1, _VCH, D // 2), lambda b, r: (b, r, 0)),
        out_shape=jax.ShapeDtypeStruct((B, LV, D // 2), jnp.int32),
    )(value, Wv, bv2, maskf)


def _sc_gather_combine(table, idxg, wg):
    mesh = plsc.VectorSubcoreMesh(core_axis_name="c", subcore_axis_name="s")

    @functools.partial(
        pl.kernel,
        mesh=mesh,
        out_type=jax.ShapeDtypeStruct((B, LQ, NH, HD), jnp.float32),
        scratch_types=[
            pltpu.VMEM((2, CQ, R), jnp.int32),
            pltpu.VMEM((2, CQ, R), jnp.float32),
            pltpu.VMEM((2, CQ, R, HD // 2), jnp.int32),
            pltpu.VMEM((CQ, HD), jnp.float32),
            pltpu.SemaphoreType.DMA,
            pltpu.SemaphoreType.DMA,
        ],
        compiler_params=pltpu.CompilerParams(use_tc_tiling_on_sc=False),
    )
    def sc_k(table_h, idx_h, w_h, out_h, idx_v, w_v, rows_v, out_v, sem0, sem1):
        sems = (sem0, sem1)
        wid = lax.axis_index("c") * 16 + lax.axis_index("s")
        b = wid // (NW // B)
        h = (wid % (NW // B)) // 2
        q0 = (wid % 2) * QW

        def stage(gg, s):
            n0 = wid * QW + gg * CQ
            pltpu.sync_copy(idx_h.at[pl.ds(n0, CQ)], idx_v.at[s])
            pltpu.sync_copy(w_h.at[pl.ds(n0, CQ)], w_v.at[s])
            for qi in range(CQ):
                pltpu.async_copy(table_h.at[idx_v.at[s, qi]],
                                 rows_v.at[s, qi], sems[s])

        def consume(gg, s):
            for qi in range(CQ):
                pltpu.make_async_copy(table_h.at[idx_v.at[s, qi]],
                                      rows_v.at[s, qi], sems[s]).wait()

            def q_body(qi, carry2):
                acc0 = jnp.zeros((16,), jnp.float32)
                acc1 = jnp.zeros((16,), jnp.float32)
                for grp in range(7):
                    base = 16 * grp if grp < 6 else R - 16
                    jstart = 0 if grp < 6 else 16 * 7 - R
                    wvec = w_v[s, qi, pl.ds(base, 16)]
                    for j in range(jstart, 16):
                        r = base + j
                        wb = jnp.full((16,), wvec[j], jnp.float32)
                        row = rows_v[s, qi, r, pl.ds(0, HD // 2)]
                        lo = lax.bitcast_convert_type(
                            row << 16, jnp.float32)
                        hi = lax.bitcast_convert_type(
                            row & jnp.int32(-65536), jnp.float32)
                        acc0 = acc0 + wb * lo
                        acc1 = acc1 + wb * hi
                out_v[qi, pl.ds(0, 16)] = acc0
                out_v[qi, pl.ds(16, 16)] = acc1
                return carry2

            lax.fori_loop(0, CQ, q_body, 0)
            pltpu.sync_copy(out_v, out_h.at[b, pl.ds(q0 + gg * CQ, CQ), h])

        stage(0, 0)

        def outer(i, carry):
            g = i * 2
            for s2 in range(2):
                gg = g + s2

                @pl.when(gg + 1 < NCHUNK)
                def _():
                    stage(gg + 1, (s2 + 1) % 2)

                consume(gg, s2)
            return carry

        lax.fori_loop(0, NCHUNK // 2, outer, 0)

    return sc_k(table, idxg, wg)


def _oproj_body(x_ref, wo_ref, bo_ref, o_ref):
    acc = lax.dot_general(x_ref[0], wo_ref[...], (((1,), (1,)), ((), ())),
                          preferred_element_type=jnp.float32)
    o_ref[0] = acc + bo_ref[...]


def _oproj_call(x, Wo, bo2):
    return pl.pallas_call(
        _oproj_body,
        grid=(B,),
        in_specs=[
            pl.BlockSpec((1, LQ, D), lambda b: (b, 0, 0)),
            pl.BlockSpec((D, D), lambda b: (0, 0)),
            pl.BlockSpec((1, D), lambda b: (0, 0)),
        ],
        out_specs=pl.BlockSpec((1, LQ, D), lambda b: (b, 0, 0)),
        out_shape=jax.ShapeDtypeStruct((B, LQ, D), jnp.float32),
    )(x, Wo, bo2)


def kernel(query, value, v_shape, v_mask, v_start_index, v_valid_ratios,
           ref_windows, Wv, bv, Wo, bo, Wbox, bbox, Wattn, battn,
           kernel_indices):
    waT = Wattn.reshape(NH, P, D)
    ba = battn.reshape(NH, 1, P)
    wbx = Wbox.reshape(NH, 5, D)
    bbx = bbox.reshape(NH, 1, 5)
    kx = kernel_indices[:, 0].reshape(1, P)
    ky = kernel_indices[:, 1].reshape(1, P)
    vr = v_valid_ratios.reshape(B, 1, 2)
    maskf = v_mask.astype(jnp.float32).reshape(B, LV, 1)

    attn4, idxg, wg = _prep_call(query, ref_windows, waT, ba, wbx, bbx,
                                 kx, ky, vr)
    vproj = _vproj_call(value, Wv[_VPERM], bv[_VPERM].reshape(1, D), maskf)
    table = vproj.reshape(B * LV * NH, HD // 2)
    sc_out = _sc_gather_combine(table, idxg.reshape(B * NH * LQ, R),
                                wg.reshape(B * NH * LQ, R))
    out = _oproj_call(sc_out.reshape(B, LQ, D), Wo, bo.reshape(1, D))
    attn_weights = attn4.reshape(B, LQ, NH, 1, 5, 5)
    return out, attn_weights


# speedup vs baseline: 11.0726x; 1.0101x over previous
import functools
import math

import numpy as np

import jax
import jax.numpy as jnp
from jax import lax
from jax.experimental import pallas as pl
from jax.experimental.pallas import tpu as pltpu
from jax.experimental.pallas import tpu_sc as plsc

B = 2
LQ = 2048
D = 256
NH = 8
HD = 32
P = 25
HF = 180
WF = 180
LV = HF * WF
R = 4 * P
NW = 32
QW = (B * NH * LQ) // NW
CQ = 16
NCHUNK = QW // CQ

_VPERM = np.concatenate([
    np.concatenate([h * HD + np.arange(HD // 2) for h in range(NH)]),
    np.concatenate([h * HD + HD // 2 + np.arange(HD // 2) for h in range(NH)]),
])


def _prep_body(q_ref, rw_ref, wa_ref, ba_ref, wb_ref, bb_ref, kx_ref, ky_ref,
               vr_ref, attn_ref, idx_ref, w_ref):
    i = pl.program_id(0)
    off = (i // NH) * (LV * NH) + (i % NH)

    qb = q_ref[0]
    wa = wa_ref[0]
    logits = lax.dot_general(qb, wa, (((1,), (1,)), ((), ())),
                             preferred_element_type=jnp.float32)
    logits = logits + ba_ref[0]
    m = jnp.max(logits, axis=-1, keepdims=True)
    e = jnp.exp(logits - m)
    attn = e / jnp.sum(e, axis=-1, keepdims=True)
    attn_ref[0, :, 0, 0] = attn

    ob = lax.dot_general(qb, wb_ref[0], (((1,), (1,)), ((), ())),
                         preferred_element_type=jnp.float32)
    ob = ob + bb_ref[0]
    rw = rw_ref[0]
    cx, cy = rw[:, 0:1], rw[:, 1:2]
    bw, bh = rw[:, 3:4], rw[:, 4:5]
    ang = rw[:, 6:7]
    dx, dy = ob[:, 0:1], ob[:, 1:2]
    dw, dh = ob[:, 2:3], ob[:, 3:4]
    da = ob[:, 4:5]

    angle = (ang + da * (1.0 / 16.0)) * (2.0 * math.pi)
    cosa = jnp.cos(angle)
    sina = jnp.sin(angle)
    ctr_x = cx + dx * (1.0 / 8.0) * bw
    ctr_y = cy + dy * (1.0 / 8.0) * bh
    sw = jnp.maximum(bw + dw * (1.0 / 8.0) * bw, 0.0)
    sh = jnp.maximum(bh + dh * (1.0 / 8.0) * bh, 0.0)

    gx = kx_ref[...] * sw
    gy = ky_ref[...] * sh
    vrx = vr_ref[0, 0:1, 0:1]
    vry = vr_ref[0, 0:1, 1:2]
    grid_x = (ctr_x + gx * cosa - gy * sina) * vrx
    grid_y = (ctr_y + gx * sina + gy * cosa) * vry

    x = grid_x * WF - 0.5
    y = grid_y * HF - 0.5
    x0 = jnp.floor(x)
    y0 = jnp.floor(y)
    lx = x - x0
    ly = y - y0
    x0i = x0.astype(jnp.int32)
    y0i = y0.astype(jnp.int32)

    def corner(yi, xi, wbil):
        valid = ((yi >= 0) & (yi < HF) & (xi >= 0) & (xi < WF))
        lin = jnp.clip(yi, 0, HF - 1) * WF + jnp.clip(xi, 0, WF - 1)
        gidx = lin * NH + off
        wc = attn * wbil * valid.astype(jnp.float32)
        return gidx, wc

    i00, w00 = corner(y0i, x0i, (1.0 - ly) * (1.0 - lx))
    i01, w01 = corner(y0i, x0i + 1, (1.0 - ly) * lx)
    i10, w10 = corner(y0i + 1, x0i, ly * (1.0 - lx))
    i11, w11 = corner(y0i + 1, x0i + 1, ly * lx)

    idx_ref[0] = jnp.concatenate([i00, i01, i10, i11], axis=1)
    w_ref[0] = jnp.concatenate([w00, w01, w10, w11], axis=1)


def _prep_call(query, ref_windows, waT, ba, wbx, bbx, kx, ky, vr):
    grid = (B * NH,)
    return pl.pallas_call(
        _prep_body,
        grid=grid,
        in_specs=[
            pl.BlockSpec((1, LQ, D), lambda i: (i // NH, 0, 0)),
            pl.BlockSpec((1, LQ, 7), lambda i: (i // NH, 0, 0)),
            pl.BlockSpec((1, P, D), lambda i: (i % NH, 0, 0)),
            pl.BlockSpec((1, 1, P), lambda i: (i % NH, 0, 0)),
            pl.BlockSpec((1, 5, D), lambda i: (i % NH, 0, 0)),
            pl.BlockSpec((1, 1, 5), lambda i: (i % NH, 0, 0)),
            pl.BlockSpec((1, P), lambda i: (0, 0)),
            pl.BlockSpec((1, P), lambda i: (0, 0)),
            pl.BlockSpec((1, 1, 2), lambda i: (i // NH, 0, 0)),
        ],
        out_specs=[
            pl.BlockSpec((1, LQ, 1, 1, P), lambda i: (i // NH, 0, i % NH, 0, 0)),
            pl.BlockSpec((1, LQ, R), lambda i: (i, 0, 0)),
            pl.BlockSpec((1, LQ, R), lambda i: (i, 0, 0)),
        ],
        out_shape=[
            jax.ShapeDtypeStruct((B, LQ, NH, 1, P), jnp.float32),
            jax.ShapeDtypeStruct((B * NH, LQ, R), jnp.int32),
            jax.ShapeDtypeStruct((B * NH, LQ, R), jnp.float32),
        ],
    )(query, ref_windows, waT, ba, wbx, bbx, kx, ky, vr)


_VCH = 3600


def _vproj_body(v_ref, wv_ref, bv_ref, m_ref, o_ref):
    acc = lax.dot_general(v_ref[0], wv_ref[...], (((1,), (1,)), ((), ())),
                          preferred_element_type=jnp.float32)
    acc = (acc + bv_ref[...]) * (1.0 - m_ref[0])
    ri = lax.bitcast_convert_type(acc, jnp.int32)
    rnd = ri + jnp.int32(0x7FFF) + ((ri >> 16) & jnp.int32(1))
    lo = (rnd[:, : D // 2] >> 16) & jnp.int32(0xFFFF)
    hi = rnd[:, D // 2:] & jnp.int32(-65536)
    o_ref[0] = lo | hi


def _vproj_call(value, Wv, bv2, maskf):
    grid = (B, LV // _VCH)
    return pl.pallas_call(
        _vproj_body,
        grid=grid,
        in_specs=[
            pl.BlockSpec((1, _VCH, D), lambda b, r: (b, r, 0)),
            pl.BlockSpec((D, D), lambda b, r: (0, 0)),
            pl.BlockSpec((1, D), lambda b, r: (0, 0)),
            pl.BlockSpec((1, _VCH, 1), lambda b, r: (b, r, 0)),
        ],
        out_specs=pl.BlockSpec((1, _VCH, D // 2), lambda b, r: (b, r, 0)),
        out_shape=jax.ShapeDtypeStruct((B, LV, D // 2), jnp.int32),
    )(value, Wv, bv2, maskf)


def _sc_gather_combine(table, idxg, wg):
    mesh = plsc.VectorSubcoreMesh(core_axis_name="c", subcore_axis_name="s")

    @functools.partial(
        pl.kernel,
        mesh=mesh,
        out_type=jax.ShapeDtypeStruct((B, LQ, NH, HD), jnp.float32),
        scratch_types=[
            pltpu.VMEM((2, CQ, R), jnp.int32),
            pltpu.VMEM((2, CQ, R), jnp.float32),
            pltpu.VMEM((2, CQ * R, HD // 2), jnp.int32),
            pltpu.VMEM((CQ, HD), jnp.float32),
            pltpu.SemaphoreType.DMA,
            pltpu.SemaphoreType.DMA,
        ],
        compiler_params=pltpu.CompilerParams(use_tc_tiling_on_sc=False),
    )
    def sc_k(table_h, idx_h, w_h, out_h, idx_v, w_v, rows_v, out_v, sem0, sem1):
        sems = (sem0, sem1)
        wid = lax.axis_index("c") * 16 + lax.axis_index("s")
        b = wid // (NW // B)
        h = (wid % (NW // B)) // 2
        q0 = (wid % 2) * QW

        def stage(gg, s):
            n0 = wid * QW + gg * CQ
            pltpu.sync_copy(idx_h.at[pl.ds(n0, CQ)], idx_v.at[s])
            pltpu.sync_copy(w_h.at[pl.ds(n0, CQ)], w_v.at[s])
            for qi in range(CQ):
                pltpu.async_copy(table_h.at[idx_v.at[s, qi]],
                                 rows_v.at[s, pl.ds(qi * R, R)], sems[s])

        def consume(gg, s):
            pltpu.make_async_copy(table_h.at[pl.ds(0, CQ * R)],
                                  rows_v.at[s], sems[s]).wait()

            def q_body(qi, carry2):
                r0 = qi * R
                na = 4
                accs = [jnp.zeros((16,), jnp.float32) for _ in range(2 * na)]
                for grp in range(7):
                    base = 16 * grp if grp < 6 else R - 16
                    jstart = 0 if grp < 6 else 16 * 7 - R
                    wvec = w_v[s, qi, pl.ds(base, 16)]
                    for j in range(jstart, 16):
                        r = base + j
                        k = r % na
                        wb = jnp.full((16,), wvec[j], jnp.float32)
                        row = rows_v[s, r0 + r, pl.ds(0, HD // 2)]
                        lo = lax.bitcast_convert_type(
                            row << 16, jnp.float32)
                        hi = lax.bitcast_convert_type(
                            row & jnp.int32(-65536), jnp.float32)
                        accs[2 * k] = accs[2 * k] + wb * lo
                        accs[2 * k + 1] = accs[2 * k + 1] + wb * hi
                acc0 = (accs[0] + accs[2]) + (accs[4] + accs[6])
                acc1 = (accs[1] + accs[3]) + (accs[5] + accs[7])
                out_v[qi, pl.ds(0, 16)] = acc0
                out_v[qi, pl.ds(16, 16)] = acc1
                return carry2

            lax.fori_loop(0, CQ, q_body, 0)
            pltpu.sync_copy(out_v, out_h.at[b, pl.ds(q0 + gg * CQ, CQ), h])

        stage(0, 0)

        def outer(i, carry):
            g = i * 2
            for s2 in range(2):
                gg = g + s2

                @pl.when(gg + 1 < NCHUNK)
                def _():
                    stage(gg + 1, (s2 + 1) % 2)

                consume(gg, s2)
            return carry

        lax.fori_loop(0, NCHUNK // 2, outer, 0)

    return sc_k(table, idxg, wg)


def _oproj_body(x_ref, wo_ref, bo_ref, o_ref):
    acc = lax.dot_general(x_ref[0], wo_ref[...], (((1,), (1,)), ((), ())),
                          preferred_element_type=jnp.float32)
    o_ref[0] = acc + bo_ref[...]


def _oproj_call(x, Wo, bo2):
    return pl.pallas_call(
        _oproj_body,
        grid=(B,),
        in_specs=[
            pl.BlockSpec((1, LQ, D), lambda b: (b, 0, 0)),
            pl.BlockSpec((D, D), lambda b: (0, 0)),
            pl.BlockSpec((1, D), lambda b: (0, 0)),
        ],
        out_specs=pl.BlockSpec((1, LQ, D), lambda b: (b, 0, 0)),
        out_shape=jax.ShapeDtypeStruct((B, LQ, D), jnp.float32),
    )(x, Wo, bo2)


def kernel(query, value, v_shape, v_mask, v_start_index, v_valid_ratios,
           ref_windows, Wv, bv, Wo, bo, Wbox, bbox, Wattn, battn,
           kernel_indices):
    waT = Wattn.reshape(NH, P, D)
    ba = battn.reshape(NH, 1, P)
    wbx = Wbox.reshape(NH, 5, D)
    bbx = bbox.reshape(NH, 1, 5)
    kx = kernel_indices[:, 0].reshape(1, P)
    ky = kernel_indices[:, 1].reshape(1, P)
    vr = v_valid_ratios.reshape(B, 1, 2)
    maskf = v_mask.astype(jnp.float32).reshape(B, LV, 1)

    attn4, idxg, wg = _prep_call(query, ref_windows, waT, ba, wbx, bbx,
                                 kx, ky, vr)
    vproj = _vproj_call(value, Wv[_VPERM], bv[_VPERM].reshape(1, D), maskf)
    table = vproj.reshape(B * LV * NH, HD // 2)
    sc_out = _sc_gather_combine(table, idxg.reshape(B * NH * LQ, R),
                                wg.reshape(B * NH * LQ, R))
    out = _oproj_call(sc_out.reshape(B, LQ, D), Wo, bo.reshape(1, D))
    attn_weights = attn4.reshape(B, LQ, NH, 1, 5, 5)
    return out, attn_weights
